# Initial kernel scaffold; baseline (speedup 1.0000x reference)
import functools

import jax
import jax.numpy as jnp
from jax import lax
from jax.experimental import pallas as pl
from jax.experimental.pallas import tpu as pltpu
from jax.experimental.pallas import tpu_sc as plsc

N = 10000
E = 320000
D = 128

NC = 2
NS = 16
NW = NC * NS

EPT = E // NW
B = 80
NCHUNK = EPT // B
RPT = N // NS


def _agg_body(h_hbm, src_hbm, dst_hbm, zeros_hbm, out_hbm,
              src_v, dst_v, rows_v, agg_sh, gsem):
    c = lax.axis_index("c")
    s = lax.axis_index("s")
    w = c * NS + s

    pltpu.sync_copy(zeros_hbm.at[pl.ds(s * RPT, RPT)],
                    agg_sh.at[pl.ds(s * RPT, RPT)])
    pltpu.sync_copy(src_hbm.at[w], src_v)
    pltpu.sync_copy(dst_hbm.at[w], dst_v)
    plsc.subcore_barrier()

    @pl.loop(0, NCHUNK)
    def _(j):
        pltpu.async_copy(h_hbm.at[src_v.at[j]], rows_v, gsem).wait()
        pltpu.sync_copy(rows_v, agg_sh.at[dst_v.at[j]], add=True)

    plsc.subcore_barrier()
    pltpu.sync_copy(agg_sh.at[pl.ds(s * RPT, RPT)],
                    out_hbm.at[c, pl.ds(s * RPT, RPT)])


_agg_call = pl.kernel(
    _agg_body,
    out_type=jax.ShapeDtypeStruct((NC, N, D), jnp.float32),
    mesh=plsc.VectorSubcoreMesh(core_axis_name="c", subcore_axis_name="s",
                                num_cores=NC, num_subcores=NS),
    scratch_types=[
        pltpu.VMEM((NCHUNK, B), jnp.int32),
        pltpu.VMEM((NCHUNK, B), jnp.int32),
        pltpu.VMEM((B, D), jnp.float32),
        pltpu.VMEM_SHARED((N, D), jnp.float32),
        pltpu.SemaphoreType.DMA,
    ],
)


ROW_BLK = 400


def _mm_body(h_ref, r0_ref, r1_ref, w_ref, b_ref, o_ref):
    rst = h_ref[...] + r0_ref[...] + r1_ref[...]
    o_ref[...] = (
        jnp.dot(rst, w_ref[...], preferred_element_type=jnp.float32)
        + b_ref[...]
    )


def _mm_call(parts, h, w, b):
    return pl.pallas_call(
        _mm_body,
        grid=(N // ROW_BLK,),
        in_specs=[
            pl.BlockSpec((ROW_BLK, D), lambda i: (i, 0)),
            pl.BlockSpec((ROW_BLK, D), lambda i: (i, 0)),
            pl.BlockSpec((ROW_BLK, D), lambda i: (i, 0)),
            pl.BlockSpec((D, D), lambda i: (0, 0)),
            pl.BlockSpec((1, D), lambda i: (0, 0)),
        ],
        out_specs=pl.BlockSpec((ROW_BLK, D), lambda i: (i, 0)),
        out_shape=jax.ShapeDtypeStruct((N, D), jnp.float32),
    )(h, parts[0], parts[1], w, b.reshape(1, D))


def kernel(h, edge_index, W0, W1, W2, W3, W4, b0, b1, b2, b3, b4):
    Ws = [W0, W1, W2, W3, W4]
    bs = [b0, b1, b2, b3, b4]
    src = edge_index[0].reshape(NW, NCHUNK, B)
    dst = edge_index[1].reshape(NW, NCHUNK, B)
    zeros = jnp.zeros((N, D), dtype=jnp.float32)
    for i in range(5):
        parts = _agg_call(h, src, dst, zeros)
        h = _mm_call(parts, h, Ws[i], bs[i])
    return h


# speedup vs baseline: 6.7034x; 6.7034x over previous
import functools

import jax
import jax.numpy as jnp
from jax import lax
from jax.experimental import pallas as pl
from jax.experimental.pallas import tpu as pltpu
from jax.experimental.pallas import tpu_sc as plsc

N = 10000
E = 320000
D = 128

NC = 2
NS = 16
NW = NC * NS

EPT = E // NW
B = 80
NCHUNK = EPT // B
NP = 10240
RPT = NP // NS


def _agg_body(h_hbm, src_hbm, dst_hbm, zeros_hbm, out_hbm,
              src_v, dst_v, rows_v, agg_sh, gsem):
    c = lax.axis_index("c")
    s = lax.axis_index("s")
    w = c * NS + s

    pltpu.sync_copy(zeros_hbm.at[pl.ds(s * RPT, RPT)],
                    agg_sh.at[pl.ds(s * RPT, RPT)])
    pltpu.sync_copy(src_hbm.at[w], src_v)
    pltpu.sync_copy(dst_hbm.at[w], dst_v)
    plsc.subcore_barrier()

    @pl.loop(0, NCHUNK)
    def _(j):
        pltpu.async_copy(h_hbm.at[src_v.at[j]], rows_v, gsem).wait()
        pltpu.sync_copy(rows_v, agg_sh.at[dst_v.at[j]], add=True)

    plsc.subcore_barrier()
    pltpu.sync_copy(agg_sh.at[pl.ds(s * RPT, RPT)],
                    out_hbm.at[c, pl.ds(s * RPT, RPT)])


_agg_call = pl.kernel(
    _agg_body,
    out_type=jax.ShapeDtypeStruct((NC, NP, D), jnp.float32),
    mesh=plsc.VectorSubcoreMesh(core_axis_name="c", subcore_axis_name="s",
                                num_cores=NC, num_subcores=NS),
    scratch_types=[
        pltpu.VMEM((NCHUNK, B), jnp.int32),
        pltpu.VMEM((NCHUNK, B), jnp.int32),
        pltpu.VMEM((B, D), jnp.float32),
        pltpu.VMEM_SHARED((NP, D), jnp.float32),
        pltpu.SemaphoreType.DMA,
    ],
)


ROW_BLK = 400


def _mm_body(h_ref, r0_ref, r1_ref, w_ref, b_ref, o_ref):
    rst = h_ref[...] + r0_ref[...] + r1_ref[...]
    o_ref[...] = (
        jnp.dot(rst, w_ref[...], preferred_element_type=jnp.float32)
        + b_ref[...]
    )


def _mm_call(parts, h, w, b):
    return pl.pallas_call(
        _mm_body,
        grid=(N // ROW_BLK,),
        in_specs=[
            pl.BlockSpec((ROW_BLK, D), lambda i: (i, 0)),
            pl.BlockSpec((ROW_BLK, D), lambda i: (i, 0)),
            pl.BlockSpec((ROW_BLK, D), lambda i: (i, 0)),
            pl.BlockSpec((D, D), lambda i: (0, 0)),
            pl.BlockSpec((1, D), lambda i: (0, 0)),
        ],
        out_specs=pl.BlockSpec((ROW_BLK, D), lambda i: (i, 0)),
        out_shape=jax.ShapeDtypeStruct((N, D), jnp.float32),
    )(h, parts[0, :N], parts[1, :N], w, b.reshape(1, D))


def kernel(h, edge_index, W0, W1, W2, W3, W4, b0, b1, b2, b3, b4):
    Ws = [W0, W1, W2, W3, W4]
    bs = [b0, b1, b2, b3, b4]
    src = edge_index[0].reshape(NW, NCHUNK, B)
    dst = edge_index[1].reshape(NW, NCHUNK, B)
    zeros = jnp.zeros((NP, D), dtype=jnp.float32)
    for i in range(5):
        parts = _agg_call(h, src, dst, zeros)
        h = _mm_call(parts, h, Ws[i], bs[i])
    return h
